# Initial kernel scaffold; baseline (speedup 1.0000x reference)
import jax
import jax.numpy as jnp
from jax.experimental import pallas as pl


def kernel(loc_data, conf_data, prior_data):
    raise NotImplementedError("write your pallas kernel here")


# speedup vs baseline: 5.1993x; 5.1993x over previous
import jax
import jax.numpy as jnp
from jax.experimental import pallas as pl
from jax.experimental.pallas import tpu as pltpu

TOP_K = 5000
CONF_THRESH = 0.01
NMS_THRESH = 0.3
VAR0, VAR1 = 0.1, 0.2
NUM_CLASSES = 2

BLK = 128
KP = 5120
NBLK = KP // BLK
KP_OUT = KP + BLK


def _col_bcast(row):
    return jnp.broadcast_to(row, (BLK, BLK)).T


def _nms_kernel(packed_ref, out_ref, scr_ref):
    g = packed_ref[0]
    s = g[0:1, :]
    lcx, lcy, lw, lh = g[1:2, :], g[2:3, :], g[3:4, :], g[4:5, :]
    pcx, pcy, pw, ph = g[5:6, :], g[6:7, :], g[7:8, :], g[8:9, :]

    cx = pcx + lcx * VAR0 * pw
    cy = pcy + lcy * VAR0 * ph
    w = pw * jnp.exp(lw * VAR1)
    h = ph * jnp.exp(lh * VAR1)
    x1 = cx - w * 0.5
    y1 = cy - h * 0.5
    x2 = x1 + w
    y2 = y1 + h
    area = (x2 - x1) * (y2 - y1)
    keep0 = jnp.where(s > CONF_THRESH, 1.0, 0.0)

    scr_ref[0:8, :] = jnp.concatenate(
        [s, x1, y1, x2, y2, area, keep0, jnp.zeros_like(s)], axis=0)
    out_ref[...] = jnp.zeros(out_ref.shape, out_ref.dtype)

    sub_i = jax.lax.broadcasted_iota(jnp.int32, (BLK, BLK), 0)
    lane_i = jax.lax.broadcasted_iota(jnp.int32, (BLK, BLK), 1)
    tri_after = sub_i < lane_i
    cum_tri = jnp.where(sub_i <= lane_i, 1.0, 0.0)
    lane_f = jax.lax.broadcasted_iota(jnp.int32, (1, BLK), 1).astype(jnp.float32)

    def blk_slice(j):
        off = pl.multiple_of(j * BLK, BLK)
        return scr_ref[0:8, pl.ds(off, BLK)]

    def process_block(j, _):
        blk = blk_slice(j)
        x1j, y1j, x2j, y2j, aj = blk[1:2], blk[2:3], blk[3:4], blk[4:5], blk[5:6]
        kb = blk[6:7]

        x1T = _col_bcast(x1j)
        y1T = _col_bcast(y1j)
        x2T = _col_bcast(x2j)
        y2T = _col_bcast(y2j)
        aT = _col_bcast(aj)

        iw = jnp.maximum(jnp.minimum(x2T, x2j) - jnp.maximum(x1T, x1j), 0.0)
        ih = jnp.maximum(jnp.minimum(y2T, y2j) - jnp.maximum(y1T, y1j), 0.0)
        inter = iw * ih
        iou = inter / (aT + aj - inter)
        S = jnp.where((iou > NMS_THRESH) & tri_after, 1.0, 0.0)

        for i in range(BLK):
            ki = kb[0, i]
            kb = kb * (1.0 - ki * S[i:i + 1, :])

        off_j = pl.multiple_of(j * BLK, BLK)
        scr_ref[6:7, pl.ds(off_j, BLK)] = kb
        kT = _col_bcast(kb)

        def sweep(l, _):
            blkl = blk_slice(l)
            x1l, y1l, x2l, y2l, al = (blkl[1:2], blkl[2:3], blkl[3:4],
                                      blkl[4:5], blkl[5:6])
            iw = jnp.maximum(jnp.minimum(x2T, x2l) - jnp.maximum(x1T, x1l), 0.0)
            ih = jnp.maximum(jnp.minimum(y2T, y2l) - jnp.maximum(y1T, y1l), 0.0)
            inter = iw * ih
            iou = inter / (aT + al - inter)
            sup = jnp.max(jnp.where(iou > NMS_THRESH, kT, 0.0), axis=0,
                          keepdims=True)
            off_l = pl.multiple_of(l * BLK, BLK)
            scr_ref[6:7, pl.ds(off_l, BLK)] = blkl[6:7] * (1.0 - sup)
            return 0

        jax.lax.fori_loop(j + 1, NBLK, sweep, 0)
        return 0

    jax.lax.fori_loop(0, NBLK, process_block, 0)

    def compact_block(j, carry):
        blk = blk_slice(j)
        kb = blk[6:7]
        incl = jnp.dot(kb, cum_tri, preferred_element_type=jnp.float32)
        carry_f = carry.astype(jnp.float32)
        rank = jnp.where(kb > 0.0, carry_f + incl - 1.0, -1e9)
        rankT = _col_bcast(rank)

        base = pl.multiple_of((carry // BLK) * BLK, BLK)
        base_f = base.astype(jnp.float32)
        e1 = jnp.where(rankT == base_f + lane_f, 1.0, 0.0)
        e2 = jnp.where(rankT == (base_f + BLK) + lane_f, 1.0, 0.0)
        c1 = jnp.dot(blk, e1, preferred_element_type=jnp.float32)
        c2 = jnp.dot(blk, e2, preferred_element_type=jnp.float32)
        out_ref[0, 0:8, pl.ds(base, BLK)] = out_ref[0, 0:8, pl.ds(base, BLK)] + c1
        b2 = pl.multiple_of(base + BLK, BLK)
        out_ref[0, 0:8, pl.ds(b2, BLK)] = out_ref[0, 0:8, pl.ds(b2, BLK)] + c2
        return carry + jnp.sum(kb).astype(jnp.int32)

    jax.lax.fori_loop(0, NBLK, compact_block, jnp.int32(0))


def kernel(loc_data, conf_data, prior_data):
    num = loc_data.shape[0]
    P = prior_data.shape[0]
    cls1 = conf_data.reshape(num, P, NUM_CLASSES)[:, :, 1]
    vals, order = jax.lax.top_k(cls1, TOP_K)

    loc_sel = jnp.take_along_axis(loc_data, order[:, :, None], axis=1)
    pri_sel = prior_data[order]
    packed = jnp.concatenate(
        [vals[:, :, None], loc_sel, pri_sel], axis=2)
    packed = jnp.transpose(packed, (0, 2, 1))
    pad_rows = jnp.zeros((num, 7, TOP_K), jnp.float32)
    packed = jnp.concatenate([packed, pad_rows], axis=1)
    lane_pad = jnp.zeros((num, 16, KP - TOP_K), jnp.float32)
    lane_pad = lane_pad.at[:, 0, :].set(-1.0)
    packed = jnp.concatenate([packed, lane_pad], axis=2)

    out = pl.pallas_call(
        _nms_kernel,
        grid=(num,),
        in_specs=[pl.BlockSpec((1, 16, KP), lambda i: (i, 0, 0))],
        out_specs=pl.BlockSpec((1, 8, KP_OUT), lambda i: (i, 0, 0)),
        out_shape=jax.ShapeDtypeStruct((num, 8, KP_OUT), jnp.float32),
        scratch_shapes=[pltpu.VMEM((8, KP), jnp.float32)],
        compiler_params=pltpu.CompilerParams(
            dimension_semantics=("parallel",)),
    )(packed)

    cls1_out = jnp.transpose(out[:, 0:5, :TOP_K], (0, 2, 1))
    bg = jnp.zeros_like(cls1_out)
    return jnp.stack([bg, cls1_out], axis=1)


# speedup vs baseline: 15.3093x; 2.9445x over previous
import jax
import jax.numpy as jnp
from jax.experimental import pallas as pl
from jax.experimental.pallas import tpu as pltpu

TOP_K = 5000
CONF_THRESH = 0.01
NMS_THRESH = 0.3
VAR0, VAR1 = 0.1, 0.2
NUM_CLASSES = 2

BLK = 128
KP = 5120
NBLK = KP // BLK
KP_OUT = KP + BLK


def _col_bcast(row):
    return jnp.broadcast_to(row, (BLK, BLK)).T


def _nms_kernel(packed_ref, out_ref, scr_ref):
    g = packed_ref[0]
    s = g[0:1, :]
    lcx, lcy, lw, lh = g[1:2, :], g[2:3, :], g[3:4, :], g[4:5, :]
    pcx, pcy, pw, ph = g[5:6, :], g[6:7, :], g[7:8, :], g[8:9, :]

    cx = pcx + lcx * VAR0 * pw
    cy = pcy + lcy * VAR0 * ph
    w = pw * jnp.exp(lw * VAR1)
    h = ph * jnp.exp(lh * VAR1)
    x1 = cx - w * 0.5
    y1 = cy - h * 0.5
    x2 = x1 + w
    y2 = y1 + h
    area = (x2 - x1) * (y2 - y1)
    keep0 = jnp.where(s > CONF_THRESH, 1.0, 0.0)

    scr_ref[0:8, :] = jnp.concatenate(
        [s, x1, y1, x2, y2, area, keep0, jnp.zeros_like(s)], axis=0)
    out_ref[...] = jnp.zeros(out_ref.shape, out_ref.dtype)

    sub_i = jax.lax.broadcasted_iota(jnp.int32, (BLK, BLK), 0)
    lane_i = jax.lax.broadcasted_iota(jnp.int32, (BLK, BLK), 1)
    tri_after = sub_i < lane_i
    cum_tri = jnp.where(sub_i <= lane_i, 1.0, 0.0)
    lane_f = jax.lax.broadcasted_iota(jnp.int32, (1, BLK), 1).astype(jnp.float32)

    def blk_slice(j):
        off = pl.multiple_of(j * BLK, BLK)
        return scr_ref[0:8, pl.ds(off, BLK)]

    def process_block(j, _):
        blk = blk_slice(j)
        x1j, y1j, x2j, y2j, aj = blk[1:2], blk[2:3], blk[3:4], blk[4:5], blk[5:6]
        kb = blk[6:7]

        x1T = _col_bcast(x1j)
        y1T = _col_bcast(y1j)
        x2T = _col_bcast(x2j)
        y2T = _col_bcast(y2j)
        aT = _col_bcast(aj)

        iw = jnp.maximum(jnp.minimum(x2T, x2j) - jnp.maximum(x1T, x1j), 0.0)
        ih = jnp.maximum(jnp.minimum(y2T, y2j) - jnp.maximum(y1T, y1j), 0.0)
        inter = iw * ih
        iou = inter / (aT + aj - inter)
        S = jnp.where((iou > NMS_THRESH) & tri_after, 1.0, 0.0)

        Sb = S.astype(jnp.bfloat16)

        def fp_cond(st):
            it, changed, _ = st
            return changed & (it < BLK + 2)

        def fp_body(st):
            it, _, x = st
            sup = jnp.dot(x.astype(jnp.bfloat16), Sb,
                          preferred_element_type=jnp.float32)
            x_new = jnp.where(sup > 0.0, 0.0, kb)
            chg = jnp.any(x_new != x)
            return it + 1, chg, x_new

        _, _, kb = jax.lax.while_loop(
            fp_cond, fp_body, (jnp.int32(0), jnp.bool_(True), kb))

        off_j = pl.multiple_of(j * BLK, BLK)
        scr_ref[6:7, pl.ds(off_j, BLK)] = kb
        kT = _col_bcast(kb)

        def sweep(l, _):
            blkl = blk_slice(l)
            x1l, y1l, x2l, y2l, al = (blkl[1:2], blkl[2:3], blkl[3:4],
                                      blkl[4:5], blkl[5:6])
            iw = jnp.maximum(jnp.minimum(x2T, x2l) - jnp.maximum(x1T, x1l), 0.0)
            ih = jnp.maximum(jnp.minimum(y2T, y2l) - jnp.maximum(y1T, y1l), 0.0)
            inter = iw * ih
            iou = inter / (aT + al - inter)
            sup = jnp.max(jnp.where(iou > NMS_THRESH, kT, 0.0), axis=0,
                          keepdims=True)
            off_l = pl.multiple_of(l * BLK, BLK)
            scr_ref[6:7, pl.ds(off_l, BLK)] = blkl[6:7] * (1.0 - sup)
            return 0

        jax.lax.fori_loop(j + 1, NBLK, sweep, 0)
        return 0

    jax.lax.fori_loop(0, NBLK, process_block, 0)

    def compact_block(j, carry):
        blk = blk_slice(j)
        kb = blk[6:7]
        incl = jnp.dot(kb, cum_tri, preferred_element_type=jnp.float32)
        carry_f = carry.astype(jnp.float32)
        rank = jnp.where(kb > 0.0, carry_f + incl - 1.0, -1e9)
        rankT = _col_bcast(rank)

        base = pl.multiple_of((carry // BLK) * BLK, BLK)
        base_f = base.astype(jnp.float32)
        e1 = jnp.where(rankT == base_f + lane_f, 1.0, 0.0)
        e2 = jnp.where(rankT == (base_f + BLK) + lane_f, 1.0, 0.0)
        c1 = jnp.dot(blk, e1, preferred_element_type=jnp.float32)
        c2 = jnp.dot(blk, e2, preferred_element_type=jnp.float32)
        out_ref[0, 0:8, pl.ds(base, BLK)] = out_ref[0, 0:8, pl.ds(base, BLK)] + c1
        b2 = pl.multiple_of(base + BLK, BLK)
        out_ref[0, 0:8, pl.ds(b2, BLK)] = out_ref[0, 0:8, pl.ds(b2, BLK)] + c2
        return carry + jnp.sum(kb).astype(jnp.int32)

    jax.lax.fori_loop(0, NBLK, compact_block, jnp.int32(0))


def kernel(loc_data, conf_data, prior_data):
    num = loc_data.shape[0]
    P = prior_data.shape[0]
    cls1 = conf_data.reshape(num, P, NUM_CLASSES)[:, :, 1]
    vals, order = jax.lax.top_k(cls1, TOP_K)

    loc_sel = jnp.take_along_axis(loc_data, order[:, :, None], axis=1)
    pri_sel = prior_data[order]
    packed = jnp.concatenate(
        [vals[:, :, None], loc_sel, pri_sel], axis=2)
    packed = jnp.transpose(packed, (0, 2, 1))
    pad_rows = jnp.zeros((num, 7, TOP_K), jnp.float32)
    packed = jnp.concatenate([packed, pad_rows], axis=1)
    lane_pad = jnp.zeros((num, 16, KP - TOP_K), jnp.float32)
    lane_pad = lane_pad.at[:, 0, :].set(-1.0)
    packed = jnp.concatenate([packed, lane_pad], axis=2)

    out = pl.pallas_call(
        _nms_kernel,
        grid=(num,),
        in_specs=[pl.BlockSpec((1, 16, KP), lambda i: (i, 0, 0))],
        out_specs=pl.BlockSpec((1, 8, KP_OUT), lambda i: (i, 0, 0)),
        out_shape=jax.ShapeDtypeStruct((num, 8, KP_OUT), jnp.float32),
        scratch_shapes=[pltpu.VMEM((8, KP), jnp.float32)],
        compiler_params=pltpu.CompilerParams(
            dimension_semantics=("parallel",)),
    )(packed)

    cls1_out = jnp.transpose(out[:, 0:5, :TOP_K], (0, 2, 1))
    bg = jnp.zeros_like(cls1_out)
    return jnp.stack([bg, cls1_out], axis=1)


# speedup vs baseline: 15.5626x; 1.0165x over previous
import jax
import jax.numpy as jnp
from jax.experimental import pallas as pl
from jax.experimental.pallas import tpu as pltpu

TOP_K = 5000
CONF_THRESH = 0.01
NMS_THRESH = 0.3
VAR0, VAR1 = 0.1, 0.2
NUM_CLASSES = 2

BLK = 128
KP = 5120
NBLK = KP // BLK
KP_OUT = KP + BLK


def _col_bcast(row):
    return jnp.broadcast_to(row, (BLK, BLK)).T


def _nms_kernel(packed_ref, out_ref, scr_ref):
    g = packed_ref[0]
    s = g[0:1, :]
    lcx, lcy, lw, lh = g[1:2, :], g[2:3, :], g[3:4, :], g[4:5, :]
    pcx, pcy, pw, ph = g[5:6, :], g[6:7, :], g[7:8, :], g[8:9, :]

    cx = pcx + lcx * VAR0 * pw
    cy = pcy + lcy * VAR0 * ph
    w = pw * jnp.exp(lw * VAR1)
    h = ph * jnp.exp(lh * VAR1)
    x1 = cx - w * 0.5
    y1 = cy - h * 0.5
    x2 = x1 + w
    y2 = y1 + h
    area = (x2 - x1) * (y2 - y1)
    keep0 = jnp.where(s > CONF_THRESH, 1.0, 0.0)

    scr_ref[0:8, 0:KP] = jnp.concatenate(
        [s, x1, y1, x2, y2, area, keep0, jnp.zeros_like(s)], axis=0)
    scr_ref[0:8, KP:KP + BLK] = jnp.zeros((8, BLK), jnp.float32)
    out_ref[...] = jnp.zeros(out_ref.shape, out_ref.dtype)

    sub_i = jax.lax.broadcasted_iota(jnp.int32, (BLK, BLK), 0)
    lane_i = jax.lax.broadcasted_iota(jnp.int32, (BLK, BLK), 1)
    tri_after = sub_i < lane_i
    cum_tri = jnp.where(sub_i <= lane_i, 1.0, 0.0)
    lane_f = jax.lax.broadcasted_iota(jnp.int32, (1, BLK), 1).astype(jnp.float32)

    def blk_slice(j):
        off = pl.multiple_of(j * BLK, BLK)
        return scr_ref[0:8, pl.ds(off, BLK)]

    def process_block(j, _):
        blk = blk_slice(j)
        x1j, y1j, x2j, y2j, aj = blk[1:2], blk[2:3], blk[3:4], blk[4:5], blk[5:6]
        kb = blk[6:7]

        x1T = _col_bcast(x1j)
        y1T = _col_bcast(y1j)
        x2T = _col_bcast(x2j)
        y2T = _col_bcast(y2j)
        aT = _col_bcast(aj)

        iw = jnp.maximum(jnp.minimum(x2T, x2j) - jnp.maximum(x1T, x1j), 0.0)
        ih = jnp.maximum(jnp.minimum(y2T, y2j) - jnp.maximum(y1T, y1j), 0.0)
        inter = iw * ih
        iou = inter / (aT + aj - inter)
        S = jnp.where((iou > NMS_THRESH) & tri_after, 1.0, 0.0)

        Sb = S.astype(jnp.bfloat16)

        def fp_cond(st):
            it, changed, _ = st
            return changed & (it < BLK + 2)

        def fp_body(st):
            it, _, x = st
            sup = jnp.dot(x.astype(jnp.bfloat16), Sb,
                          preferred_element_type=jnp.float32)
            x_new = jnp.where(sup > 0.0, 0.0, kb)
            chg = jnp.any(x_new != x)
            return it + 1, chg, x_new

        _, _, kb = jax.lax.while_loop(
            fp_cond, fp_body, (jnp.int32(0), jnp.bool_(True), kb))

        off_j = pl.multiple_of(j * BLK, BLK)
        scr_ref[6:7, pl.ds(off_j, BLK)] = kb
        kT = _col_bcast(kb)

        x1T2 = pltpu.repeat(x1T, 2, axis=1)
        y1T2 = pltpu.repeat(y1T, 2, axis=1)
        x2T2 = pltpu.repeat(x2T, 2, axis=1)
        y2T2 = pltpu.repeat(y2T, 2, axis=1)
        aT2 = pltpu.repeat(aT, 2, axis=1)
        kT2 = pltpu.repeat(kT, 2, axis=1)

        def sweep(t, _):
            off_l = pl.multiple_of((j + 1 + 2 * t) * BLK, BLK)
            blkl = scr_ref[0:8, pl.ds(off_l, 2 * BLK)]
            x1l, y1l, x2l, y2l, al = (blkl[1:2], blkl[2:3], blkl[3:4],
                                      blkl[4:5], blkl[5:6])
            iw = jnp.maximum(jnp.minimum(x2T2, x2l) - jnp.maximum(x1T2, x1l), 0.0)
            ih = jnp.maximum(jnp.minimum(y2T2, y2l) - jnp.maximum(y1T2, y1l), 0.0)
            inter = iw * ih
            iou = inter / (aT2 + al - inter)
            sup = jnp.max(jnp.where(iou > NMS_THRESH, kT2, 0.0), axis=0,
                          keepdims=True)
            scr_ref[6:7, pl.ds(off_l, 2 * BLK)] = blkl[6:7] * (1.0 - sup)
            return 0

        npairs = (NBLK - j) // 2
        jax.lax.fori_loop(0, npairs, sweep, 0)
        return 0

    jax.lax.fori_loop(0, NBLK, process_block, 0)

    def compact_block(j, carry):
        blk = blk_slice(j)
        kb = blk[6:7]
        incl = jnp.dot(kb, cum_tri, preferred_element_type=jnp.float32)
        carry_f = carry.astype(jnp.float32)
        rank = jnp.where(kb > 0.0, carry_f + incl - 1.0, -1e9)
        rankT = _col_bcast(rank)

        base = pl.multiple_of((carry // BLK) * BLK, BLK)
        base_f = base.astype(jnp.float32)
        e1 = jnp.where(rankT == base_f + lane_f, 1.0, 0.0)
        e2 = jnp.where(rankT == (base_f + BLK) + lane_f, 1.0, 0.0)
        c1 = jnp.dot(blk, e1, preferred_element_type=jnp.float32)
        c2 = jnp.dot(blk, e2, preferred_element_type=jnp.float32)
        out_ref[0, 0:8, pl.ds(base, BLK)] = out_ref[0, 0:8, pl.ds(base, BLK)] + c1
        b2 = pl.multiple_of(base + BLK, BLK)
        out_ref[0, 0:8, pl.ds(b2, BLK)] = out_ref[0, 0:8, pl.ds(b2, BLK)] + c2
        return carry + jnp.sum(kb).astype(jnp.int32)

    jax.lax.fori_loop(0, NBLK, compact_block, jnp.int32(0))


def kernel(loc_data, conf_data, prior_data):
    num = loc_data.shape[0]
    P = prior_data.shape[0]
    cls1 = conf_data.reshape(num, P, NUM_CLASSES)[:, :, 1]
    vals, order = jax.lax.top_k(cls1, TOP_K)

    loc_sel = jnp.take_along_axis(loc_data, order[:, :, None], axis=1)
    pri_sel = prior_data[order]
    packed = jnp.concatenate(
        [vals[:, :, None], loc_sel, pri_sel], axis=2)
    packed = jnp.transpose(packed, (0, 2, 1))
    pad_rows = jnp.zeros((num, 7, TOP_K), jnp.float32)
    packed = jnp.concatenate([packed, pad_rows], axis=1)
    lane_pad = jnp.zeros((num, 16, KP - TOP_K), jnp.float32)
    lane_pad = lane_pad.at[:, 0, :].set(-1.0)
    packed = jnp.concatenate([packed, lane_pad], axis=2)

    out = pl.pallas_call(
        _nms_kernel,
        grid=(num,),
        in_specs=[pl.BlockSpec((1, 16, KP), lambda i: (i, 0, 0))],
        out_specs=pl.BlockSpec((1, 8, KP_OUT), lambda i: (i, 0, 0)),
        out_shape=jax.ShapeDtypeStruct((num, 8, KP_OUT), jnp.float32),
        scratch_shapes=[pltpu.VMEM((8, KP + BLK), jnp.float32)],
        compiler_params=pltpu.CompilerParams(
            dimension_semantics=("parallel",)),
    )(packed)

    cls1_out = jnp.transpose(out[:, 0:5, :TOP_K], (0, 2, 1))
    bg = jnp.zeros_like(cls1_out)
    return jnp.stack([bg, cls1_out], axis=1)


# speedup vs baseline: 15.5844x; 1.0014x over previous
import jax
import jax.numpy as jnp
from jax.experimental import pallas as pl
from jax.experimental.pallas import tpu as pltpu

TOP_K = 5000
CONF_THRESH = 0.01
NMS_THRESH = 0.3
VAR0, VAR1 = 0.1, 0.2
NUM_CLASSES = 2

BLK = 128
KP = 5120
NBLK = KP // BLK
KP_OUT = KP + BLK


def _col_bcast(row):
    return jnp.broadcast_to(row, (BLK, BLK)).T


def _nms_kernel(packed_ref, out_ref, scr_ref):
    g = packed_ref[0]
    s = g[0:1, :]
    lcx, lcy, lw, lh = g[1:2, :], g[2:3, :], g[3:4, :], g[4:5, :]
    pcx, pcy, pw, ph = g[5:6, :], g[6:7, :], g[7:8, :], g[8:9, :]

    cx = pcx + lcx * VAR0 * pw
    cy = pcy + lcy * VAR0 * ph
    w = pw * jnp.exp(lw * VAR1)
    h = ph * jnp.exp(lh * VAR1)
    x1 = cx - w * 0.5
    y1 = cy - h * 0.5
    x2 = x1 + w
    y2 = y1 + h
    area = (x2 - x1) * (y2 - y1)
    keep0 = jnp.where(s > CONF_THRESH, 1.0, 0.0)

    scr_ref[0:8, 0:KP] = jnp.concatenate(
        [s, x1, y1, x2, y2, area, keep0, jnp.zeros_like(s)], axis=0)
    scr_ref[0:8, KP:KP + BLK] = jnp.zeros((8, BLK), jnp.float32)
    out_ref[...] = jnp.zeros(out_ref.shape, out_ref.dtype)

    sub_i = jax.lax.broadcasted_iota(jnp.int32, (BLK, BLK), 0)
    lane_i = jax.lax.broadcasted_iota(jnp.int32, (BLK, BLK), 1)
    tri_after = sub_i < lane_i
    cum_tri = jnp.where(sub_i <= lane_i, 1.0, 0.0)
    lane2_f = jax.lax.broadcasted_iota(
        jnp.int32, (1, 2 * BLK), 1).astype(jnp.float32)

    def blk_slice(j):
        off = pl.multiple_of(j * BLK, BLK)
        return scr_ref[0:8, pl.ds(off, BLK)]

    def process_block(j, _):
        blk = blk_slice(j)
        x1j, y1j, x2j, y2j, aj = blk[1:2], blk[2:3], blk[3:4], blk[4:5], blk[5:6]
        kb = blk[6:7]

        x1T = _col_bcast(x1j)
        y1T = _col_bcast(y1j)
        x2T = _col_bcast(x2j)
        y2T = _col_bcast(y2j)
        aT = _col_bcast(aj)

        iw = jnp.maximum(jnp.minimum(x2T, x2j) - jnp.maximum(x1T, x1j), 0.0)
        ih = jnp.maximum(jnp.minimum(y2T, y2j) - jnp.maximum(y1T, y1j), 0.0)
        inter = iw * ih
        iou = inter / (aT + aj - inter)
        S = jnp.where((iou > NMS_THRESH) & tri_after, 1.0, 0.0)

        Sb = S.astype(jnp.bfloat16)

        def fp_cond(st):
            it, changed, _ = st
            return changed & (it < BLK + 2)

        def fp_body(st):
            it, _, x = st
            sup = jnp.dot(x.astype(jnp.bfloat16), Sb,
                          preferred_element_type=jnp.float32)
            x_new = jnp.where(sup > 0.0, 0.0, kb)
            chg = jnp.any(x_new != x)
            return it + 1, chg, x_new

        _, _, kb = jax.lax.while_loop(
            fp_cond, fp_body, (jnp.int32(0), jnp.bool_(True), kb))

        off_j = pl.multiple_of(j * BLK, BLK)
        scr_ref[6:7, pl.ds(off_j, BLK)] = kb
        kT = _col_bcast(kb)

        x1T2 = pltpu.repeat(x1T, 2, axis=1)
        y1T2 = pltpu.repeat(y1T, 2, axis=1)
        x2T2 = pltpu.repeat(x2T, 2, axis=1)
        y2T2 = pltpu.repeat(y2T, 2, axis=1)
        aT2 = pltpu.repeat(aT, 2, axis=1)
        kT2 = pltpu.repeat(kT, 2, axis=1)

        def sweep(t, _):
            off_l = pl.multiple_of((j + 1 + 2 * t) * BLK, BLK)
            blkl = scr_ref[0:8, pl.ds(off_l, 2 * BLK)]
            x1l, y1l, x2l, y2l, al = (blkl[1:2], blkl[2:3], blkl[3:4],
                                      blkl[4:5], blkl[5:6])
            iw = jnp.maximum(jnp.minimum(x2T2, x2l) - jnp.maximum(x1T2, x1l), 0.0)
            ih = jnp.maximum(jnp.minimum(y2T2, y2l) - jnp.maximum(y1T2, y1l), 0.0)
            inter = iw * ih
            iou = inter / (aT2 + al - inter)
            sup = jnp.max(jnp.where(iou > NMS_THRESH, kT2, 0.0), axis=0,
                          keepdims=True)
            scr_ref[6:7, pl.ds(off_l, 2 * BLK)] = blkl[6:7] * (1.0 - sup)
            return 0

        npairs = (NBLK - j) // 2
        jax.lax.fori_loop(0, npairs, sweep, 0)
        return 0

    jax.lax.fori_loop(0, NBLK, process_block, 0)

    def compact_block(j, carry):
        blk = blk_slice(j)
        kb = blk[6:7]
        incl = jnp.dot(kb, cum_tri, preferred_element_type=jnp.float32)
        carry_f = carry.astype(jnp.float32)
        rank = jnp.where(kb > 0.0, carry_f + incl - 1.0, -1e9)
        rankT = _col_bcast(rank)

        base = pl.multiple_of((carry // BLK) * BLK, BLK)
        base_f = base.astype(jnp.float32)
        rankT2 = pltpu.repeat(rankT, 2, axis=1)
        e = jnp.where(rankT2 == base_f + lane2_f, 1.0, 0.0)
        c = jnp.dot(blk, e, preferred_element_type=jnp.float32)
        out_ref[0, 0:8, pl.ds(base, 2 * BLK)] = (
            out_ref[0, 0:8, pl.ds(base, 2 * BLK)] + c)
        return carry + jnp.sum(kb).astype(jnp.int32)

    jax.lax.fori_loop(0, NBLK, compact_block, jnp.int32(0))


def kernel(loc_data, conf_data, prior_data):
    num = loc_data.shape[0]
    P = prior_data.shape[0]
    cls1 = conf_data.reshape(num, P, NUM_CLASSES)[:, :, 1]
    vals, order = jax.lax.top_k(cls1, TOP_K)

    loc_sel = jnp.take_along_axis(loc_data, order[:, :, None], axis=1)
    pri_sel = prior_data[order]
    packed = jnp.concatenate(
        [vals[:, :, None], loc_sel, pri_sel], axis=2)
    packed = jnp.transpose(packed, (0, 2, 1))
    pad_rows = jnp.zeros((num, 7, TOP_K), jnp.float32)
    packed = jnp.concatenate([packed, pad_rows], axis=1)
    lane_pad = jnp.zeros((num, 16, KP - TOP_K), jnp.float32)
    lane_pad = lane_pad.at[:, 0, :].set(-1.0)
    packed = jnp.concatenate([packed, lane_pad], axis=2)

    out = pl.pallas_call(
        _nms_kernel,
        grid=(num,),
        in_specs=[pl.BlockSpec((1, 16, KP), lambda i: (i, 0, 0))],
        out_specs=pl.BlockSpec((1, 8, KP_OUT), lambda i: (i, 0, 0)),
        out_shape=jax.ShapeDtypeStruct((num, 8, KP_OUT), jnp.float32),
        scratch_shapes=[pltpu.VMEM((8, KP + BLK), jnp.float32)],
        compiler_params=pltpu.CompilerParams(
            dimension_semantics=("parallel",)),
    )(packed)

    cls1_out = jnp.transpose(out[:, 0:5, :TOP_K], (0, 2, 1))
    bg = jnp.zeros_like(cls1_out)
    return jnp.stack([bg, cls1_out], axis=1)
